# baseline (speedup 1.0000x reference)
import functools

import jax
import jax.numpy as jnp
from jax import lax
from jax.experimental import pallas as pl
from jax.experimental.pallas import tpu as pltpu

B = 16384
BLK = 4096
D = 32
V_CENTER, V_SUBJECT, V_GRADE, V_METHOD = 66, 34, 18, 10
OFF_S = V_CENTER
OFF_G = OFF_S + V_SUBJECT
OFF_M = OFF_G + V_GRADE
NCAT = OFF_M + V_METHOD
TEXP_ROWS = 136


def _body(c_ref, s_ref, g_ref, m_ref, ct_ref,
          texp_ref, w1_ref, b1_ref, w2_ref, b2_ref, w3_ref, b3_ref,
          out_ref, m_scr):
    @pl.when(pl.program_id(0) == 0)
    def _fold():
        m_scr[...] = jnp.dot(texp_ref[...], w1_ref[...],
                             preferred_element_type=jnp.float32)

    c = c_ref[0, 0, :]
    s = s_ref[0, 0, :] + OFF_S
    g = g_ref[0, 0, :] + OFF_G
    m = m_ref[0, 0, :] + OFF_M

    col = lax.broadcasted_iota(jnp.int32, (BLK, NCAT), 1)
    onehot = ((col == c[:, None])
              | (col == s[:, None])
              | (col == g[:, None])
              | (col == m[:, None])).astype(jnp.float32)

    mcat = m_scr[0:NCAT, :]
    cwtw = m_scr[NCAT:NCAT + 2, :]
    b1pp = b1_ref[...] + m_scr[NCAT + 2:NCAT + 3, :] + m_scr[NCAT + 3:NCAT + 4, :]

    h1 = jnp.dot(onehot, mcat, preferred_element_type=jnp.float32)
    h1 = h1 + lax.dot_general(ct_ref[0], cwtw, (((0,), (0,)), ((), ())),
                              preferred_element_type=jnp.float32)
    h1 = jnp.maximum(h1 + b1pp, 0.0)
    h2 = jnp.maximum(
        jnp.dot(h1, w2_ref[...], preferred_element_type=jnp.float32) + b2_ref[...],
        0.0)
    out_ref[...] = (jnp.dot(h2, w3_ref[...], preferred_element_type=jnp.float32)
                    + b3_ref[...])


def kernel(center_idx, subject_idx, grade_idx, method_idx, cost, time,
           center_table, subject_table, grade_table, method_table,
           cost_W, cost_b, time_W, time_b,
           W1, b1, W2, b2, W3, b3):
    nb = B // BLK
    r3 = lambda x: x.astype(jnp.int32).reshape(nb, 1, BLK)
    c3, s3, g3, m3 = r3(center_idx), r3(subject_idx), r3(grade_idx), r3(method_idx)
    ct3 = jnp.stack([cost, time], axis=0).reshape(2, nb, BLK).transpose(1, 0, 2)

    texp = jnp.zeros((TEXP_ROWS, 192), dtype=jnp.float32)
    texp = texp.at[0:OFF_S, 0:32].set(center_table)
    texp = texp.at[OFF_S:OFF_G, 32:64].set(subject_table)
    texp = texp.at[OFF_G:OFF_M, 64:96].set(grade_table)
    texp = texp.at[OFF_M:NCAT, 96:128].set(method_table)
    texp = texp.at[NCAT, 128:160].set(cost_W[0])
    texp = texp.at[NCAT + 1, 160:192].set(time_W[0])
    texp = texp.at[NCAT + 2, 128:160].set(cost_b)
    texp = texp.at[NCAT + 3, 160:192].set(time_b)

    idx_spec = pl.BlockSpec((1, 1, BLK), lambda i: (i, 0, 0))
    ct_spec = pl.BlockSpec((1, 2, BLK), lambda i: (i, 0, 0))
    full = lambda a: pl.BlockSpec(a.shape, lambda i: (0,) * a.ndim)

    b1r, b2r, b3r = b1.reshape(1, 256), b2.reshape(1, 128), b3.reshape(1, 32)

    return pl.pallas_call(
        _body,
        grid=(nb,),
        in_specs=[idx_spec, idx_spec, idx_spec, idx_spec, ct_spec,
                  full(texp), full(W1), full(b1r), full(W2), full(b2r),
                  full(W3), full(b3r)],
        out_specs=pl.BlockSpec((BLK, D), lambda i: (i, 0)),
        out_shape=jax.ShapeDtypeStruct((B, D), jnp.float32),
        scratch_shapes=[pltpu.VMEM((TEXP_ROWS, 256), jnp.float32)],
        compiler_params=pltpu.CompilerParams(
            dimension_semantics=("arbitrary",)),
    )(c3, s3, g3, m3, ct3, texp, W1, b1r, W2, b2r, W3, b3r)


# speedup vs baseline: 1.0329x; 1.0329x over previous
import functools

import jax
import jax.numpy as jnp
from jax import lax
from jax.experimental import pallas as pl
from jax.experimental.pallas import tpu as pltpu

B = 16384
BLK = 4096
D = 32
V_CENTER, V_SUBJECT, V_GRADE, V_METHOD = 66, 34, 18, 10
OFF_S = V_CENTER
OFF_G = OFF_S + V_SUBJECT
OFF_M = OFF_G + V_GRADE
NCAT = OFF_M + V_METHOD
TEXP_ROWS = 136


def _body(c_ref, s_ref, g_ref, m_ref, cost_ref, time_ref,
          texp_ref, w1_ref, b1_ref, w2_ref, b2_ref, w3_ref, b3_ref,
          out_ref, m_scr):
    @pl.when(pl.program_id(0) == 0)
    def _fold():
        m_scr[...] = jnp.dot(texp_ref[...], w1_ref[...],
                             preferred_element_type=jnp.float32)

    c = c_ref[...]
    s = s_ref[...] + OFF_S
    g = g_ref[...] + OFF_G
    m = m_ref[...] + OFF_M

    col = lax.broadcasted_iota(jnp.int32, (BLK, NCAT), 1)
    onehot = ((col == c[:, None])
              | (col == s[:, None])
              | (col == g[:, None])
              | (col == m[:, None])).astype(jnp.float32)

    ct = jnp.stack([cost_ref[...], time_ref[...]], axis=0)

    mcat = m_scr[0:NCAT, :]
    cwtw = m_scr[NCAT:NCAT + 2, :]
    b1pp = (b1_ref[...][None, :] + m_scr[NCAT + 2:NCAT + 3, :]
            + m_scr[NCAT + 3:NCAT + 4, :])

    h1 = jnp.dot(onehot, mcat, preferred_element_type=jnp.float32)
    h1 = h1 + lax.dot_general(ct, cwtw, (((0,), (0,)), ((), ())),
                              preferred_element_type=jnp.float32)
    h1 = jnp.maximum(h1 + b1pp, 0.0)
    h2 = jnp.maximum(
        jnp.dot(h1, w2_ref[...], preferred_element_type=jnp.float32)
        + b2_ref[...][None, :],
        0.0)
    out_ref[...] = (jnp.dot(h2, w3_ref[...], preferred_element_type=jnp.float32)
                    + b3_ref[...][None, :])


def kernel(center_idx, subject_idx, grade_idx, method_idx, cost, time,
           center_table, subject_table, grade_table, method_table,
           cost_W, cost_b, time_W, time_b,
           W1, b1, W2, b2, W3, b3):
    nb = B // BLK
    ci = center_idx.astype(jnp.int32)
    si = subject_idx.astype(jnp.int32)
    gi = grade_idx.astype(jnp.int32)
    mi = method_idx.astype(jnp.int32)

    texp = jnp.zeros((TEXP_ROWS, 192), dtype=jnp.float32)
    texp = texp.at[0:OFF_S, 0:32].set(center_table)
    texp = texp.at[OFF_S:OFF_G, 32:64].set(subject_table)
    texp = texp.at[OFF_G:OFF_M, 64:96].set(grade_table)
    texp = texp.at[OFF_M:NCAT, 96:128].set(method_table)
    texp = texp.at[NCAT, 128:160].set(cost_W[0])
    texp = texp.at[NCAT + 1, 160:192].set(time_W[0])
    texp = texp.at[NCAT + 2, 128:160].set(cost_b)
    texp = texp.at[NCAT + 3, 160:192].set(time_b)

    vec_spec = pl.BlockSpec((BLK,), lambda i: (i,))
    full = lambda a: pl.BlockSpec(a.shape, lambda i: (0,) * a.ndim)

    return pl.pallas_call(
        _body,
        grid=(nb,),
        in_specs=[vec_spec, vec_spec, vec_spec, vec_spec, vec_spec, vec_spec,
                  full(texp), full(W1), full(b1), full(W2), full(b2),
                  full(W3), full(b3)],
        out_specs=pl.BlockSpec((BLK, D), lambda i: (i, 0)),
        out_shape=jax.ShapeDtypeStruct((B, D), jnp.float32),
        scratch_shapes=[pltpu.VMEM((TEXP_ROWS, 256), jnp.float32)],
        compiler_params=pltpu.CompilerParams(
            dimension_semantics=("arbitrary",)),
    )(ci, si, gi, mi, cost, time, texp, W1, b1, W2, b2, W3, b3)
